# baseline (speedup 1.0000x reference)
import functools

import jax
import jax.numpy as jnp
from jax import lax
from jax.experimental import pallas as pl
from jax.experimental.pallas import tpu as pltpu
from jax.experimental.pallas import tpu_sc as plsc

B, T, N, C = 4, 4, 1024, 64
K = 16
NP = 512
F32 = jnp.float32


def _fps_body(xt_ref, aidx_ref, ax_ref, ay_ref, az_ref, dists_ref):
    x = xt_ref[0]
    y = xt_ref[1]
    z = xt_ref[2]
    iota = lax.broadcasted_iota(jnp.int32, (B * T, N), 1).astype(F32)
    step_iota = lax.broadcasted_iota(jnp.int32, (B * T, NP), 1).astype(F32)
    dists_ref[...] = jnp.full((B * T, N), 1e10, F32)
    aidx_ref[...] = jnp.zeros((B * T, NP), F32)
    ax_ref[...] = jnp.zeros((B * T, NP), F32)
    ay_ref[...] = jnp.zeros((B * T, NP), F32)
    az_ref[...] = jnp.zeros((B * T, NP), F32)

    def step(i, far):
        hot = step_iota == i.astype(F32)
        aidx_ref[...] += jnp.where(hot, far, 0.0)
        sel = iota == far
        cx = jnp.sum(jnp.where(sel, x, 0.0), axis=1, keepdims=True)
        cy = jnp.sum(jnp.where(sel, y, 0.0), axis=1, keepdims=True)
        cz = jnp.sum(jnp.where(sel, z, 0.0), axis=1, keepdims=True)
        ax_ref[...] += jnp.where(hot, cx, 0.0)
        ay_ref[...] += jnp.where(hot, cy, 0.0)
        az_ref[...] += jnp.where(hot, cz, 0.0)
        dx = x - cx
        dy = y - cy
        dz = z - cz
        d = (dx * dx + dy * dy) + dz * dz
        dmin = jnp.minimum(dists_ref[...], d)
        dists_ref[...] = dmin
        m = jnp.max(dmin, axis=1, keepdims=True)
        far2 = jnp.min(jnp.where(dmin == m, iota, F32(1e9)), axis=1, keepdims=True)
        return far2

    lax.fori_loop(0, NP, step, jnp.zeros((B * T, 1), F32))


def _run_fps(xyzs):
    xt = jnp.transpose(xyzs.reshape(B * T, N, 3), (2, 0, 1))
    shp = jax.ShapeDtypeStruct((B * T, NP), F32)
    spec = pl.BlockSpec((B * T, NP), lambda: (0, 0))
    return pl.pallas_call(
        _fps_body,
        out_shape=(shp, shp, shp, shp),
        in_specs=[pl.BlockSpec((3, B * T, N), lambda: (0, 0, 0))],
        out_specs=(spec, spec, spec, spec),
        scratch_shapes=[pltpu.VMEM((B * T, N), F32)],
    )(xt)


L = 16


def _fps_sc_body(x_hbm, y_hbm, z_hbm, aidx_hbm, ax_hbm, ay_hbm, az_hbm,
                 x_v, y_v, z_v, dmin_v, aidx_v, ax_v, ay_v, az_v):
    nc = 2
    wid = lax.axis_index("s") * nc + lax.axis_index("c")

    @pl.when(wid < B * T)
    def _():
        pltpu.sync_copy(x_hbm.at[pl.ds(wid * N, N)], x_v)
        pltpu.sync_copy(y_hbm.at[pl.ds(wid * N, N)], y_v)
        pltpu.sync_copy(z_hbm.at[pl.ds(wid * N, N)], z_v)
        lane_i = lax.iota(jnp.int32, L)
        lane_f = lane_i.astype(F32)
        big = F32(1e30)

        def fold(v, op):
            for sh in (8, 4, 2, 1):
                v = op(v, v.at[lane_i ^ sh].get(mode="promise_in_bounds"))
            return v

        for c in range(N // L):
            dmin_v[pl.ds(c * L, L)] = jnp.full((L,), 1e10, F32)

        cx0 = fold(jnp.where(lane_i == 0, x_v[pl.ds(0, L)], big), jnp.minimum)
        cy0 = fold(jnp.where(lane_i == 0, y_v[pl.ds(0, L)], big), jnp.minimum)
        cz0 = fold(jnp.where(lane_i == 0, z_v[pl.ds(0, L)], big), jnp.minimum)

        def step(i, carry):
            cx, cy, cz, farf, bA, bX, bY, bZ = carry
            sel = lane_i == (i % L)
            bA = jnp.where(sel, farf, bA)
            bX = jnp.where(sel, cx, bX)
            bY = jnp.where(sel, cy, bY)
            bZ = jnp.where(sel, cz, bZ)

            @pl.when(i % L == L - 1)
            def _flush():
                base = i - (L - 1)
                aidx_v[pl.ds(base, L)] = bA
                ax_v[pl.ds(base, L)] = bX
                ay_v[pl.ds(base, L)] = bY
                az_v[pl.ds(base, L)] = bZ

            bestv = jnp.full((L,), -1.0, F32)
            besti = jnp.full((L,), 0.0, F32)
            bestx = jnp.zeros((L,), F32)
            besty = jnp.zeros((L,), F32)
            bestz = jnp.zeros((L,), F32)
            for c in range(N // L):
                sl = pl.ds(c * L, L)
                xc = x_v[sl]
                yc = y_v[sl]
                zc = z_v[sl]
                dx = xc - cx
                dy = yc - cy
                dz = zc - cz
                d = (dx * dx + dy * dy) + dz * dz
                nm = jnp.minimum(dmin_v[sl], d)
                dmin_v[sl] = nm
                upd = nm > bestv
                bestv = jnp.where(upd, nm, bestv)
                besti = jnp.where(upd, lane_f + F32(c * L), besti)
                bestx = jnp.where(upd, xc, bestx)
                besty = jnp.where(upd, yc, besty)
                bestz = jnp.where(upd, zc, bestz)
            m = fold(bestv, jnp.maximum)
            far2 = fold(jnp.where(bestv == m, besti, F32(1e9)), jnp.minimum)
            win = besti == far2
            ncx = fold(jnp.where(win, bestx, big), jnp.minimum)
            ncy = fold(jnp.where(win, besty, big), jnp.minimum)
            ncz = fold(jnp.where(win, bestz, big), jnp.minimum)
            return (ncx, ncy, ncz, far2, bA, bX, bY, bZ)

        zz = jnp.zeros((L,), F32)
        init = (cx0, cy0, cz0, zz, zz, zz, zz, zz)
        lax.fori_loop(0, NP, step, init)

        pltpu.sync_copy(aidx_v, aidx_hbm.at[pl.ds(wid * NP, NP)])
        pltpu.sync_copy(ax_v, ax_hbm.at[pl.ds(wid * NP, NP)])
        pltpu.sync_copy(ay_v, ay_hbm.at[pl.ds(wid * NP, NP)])
        pltpu.sync_copy(az_v, az_hbm.at[pl.ds(wid * NP, NP)])


def _run_fps_sc(xyzs):
    xt = jnp.transpose(xyzs.reshape(B * T, N, 3), (2, 0, 1))
    xf = xt[0].reshape(B * T * N)
    yf = xt[1].reshape(B * T * N)
    zf = xt[2].reshape(B * T * N)
    shp = jax.ShapeDtypeStruct((B * T * NP,), F32)
    fn = pl.kernel(
        _fps_sc_body,
        out_type=(shp, shp, shp, shp),
        mesh=plsc.VectorSubcoreMesh(core_axis_name="c", subcore_axis_name="s"),
        scratch_types=[
            pltpu.VMEM((N,), F32),
            pltpu.VMEM((N,), F32),
            pltpu.VMEM((N,), F32),
            pltpu.VMEM((N,), F32),
            pltpu.VMEM((NP,), F32),
            pltpu.VMEM((NP,), F32),
            pltpu.VMEM((NP,), F32),
            pltpu.VMEM((NP,), F32),
        ],
    )
    aidx, ax, ay, az = fn(xf, yf, zf)
    rs = lambda a: a.reshape(B * T, NP)
    return rs(aidx), rs(ax), rs(ay), rs(az)


def _main_body(xta_ref, xtn_ref, fa_ref, fn_ref, aidx_ref,
               ax_ref, ay_ref, az_ref,
               w1x_ref, w1f_ref, w1a_ref, b1_ref, w2_ref, b2_ref,
               nxyz_ref, nfeat_ref):
    xa = xta_ref[0, 0]
    xn = xtn_ref[0, 0]
    f1 = fa_ref[0, 0]
    f2 = fn_ref[0, 0]
    aidx = aidx_ref[0]
    ax = ax_ref[0]
    ay = ay_ref[0]
    az = az_ref[0]
    w1x = w1x_ref[...]
    w1f = w1f_ref[...]
    w1a = w1a_ref[...]
    b1 = b1_ref[...]
    w2 = w2_ref[...]
    b2 = b2_ref[...]

    iota = lax.broadcasted_iota(jnp.int32, (NP, N), 1).astype(F32)
    oh_a = (iota == aidx).astype(F32)
    nxyz_ref[0, 0] = jnp.concatenate([ax, ay, az], axis=1)

    xyz1 = jnp.transpose(xa)
    xyz2 = jnp.transpose(xn)
    dot = functools.partial(jnp.dot, preferred_element_type=F32)
    q1 = dot(f1, w1a) - dot(xyz1, w1x)
    a_mat = dot(oh_a, q1) + b1
    p_mat = dot(f2, w1f) + dot(xyz2, w1x)

    dx = ax - xn[0:1, :]
    dy = ay - xn[1:2, :]
    dz = az - xn[2:3, :]
    s = (dx * dx + dy * dy) + dz * dz

    ones_col = jnp.ones((N, 1), F32)
    p16 = p_mat.astype(jnp.bfloat16)
    w216 = w2.astype(jnp.bfloat16)
    acc = None
    for _ in range(K):
        mval = jnp.min(s, axis=1, keepdims=True)
        eq = s == mval
        oh = jnp.where(eq, F32(1.0), F32(0.0))
        cnt = dot(oh, ones_col)
        oh16 = oh.astype(jnp.bfloat16)
        g = dot(oh16, p16) * (1.0 / cnt)
        h = jnp.maximum(g + a_mat, 0.0)
        o = jnp.maximum(dot(h.astype(jnp.bfloat16), w216) + b2, 0.0)
        acc = o if acc is None else jnp.maximum(acc, o)
        s = jnp.where(eq, F32(3e38), s)

    nfeat_ref[0, 0] = acc


def kernel(xyzs, features, W1, b1, W2, b2):
    aidx, axc, ayc, azc = _run_fps_sc(xyzs)
    aidx3 = aidx.reshape(B * T, NP, 1)
    ax3 = axc.reshape(B * T, NP, 1)
    ay3 = ayc.reshape(B * T, NP, 1)
    az3 = azc.reshape(B * T, NP, 1)

    xt = jnp.transpose(xyzs, (0, 1, 3, 2))
    w1x = W1[0:3]
    w1f = W1[3:3 + C]
    w1a = W1[3 + C:3 + 2 * C]
    b1r = b1.reshape(1, C)
    b2r = b2.reshape(1, 2 * C)

    t_last = T - 1
    nb = lambda b, t: (b, jnp.minimum(t + 1, t_last), 0, 0)

    out_shapes = (
        jax.ShapeDtypeStruct((B, T, NP, 3), F32),
        jax.ShapeDtypeStruct((B, T, NP, 2 * C), F32),
    )
    grid = (B, T)
    new_xyzs, new_feats = pl.pallas_call(
        _main_body,
        grid=grid,
        out_shape=out_shapes,
        in_specs=[
            pl.BlockSpec((1, 1, 3, N), lambda b, t: (b, t, 0, 0)),
            pl.BlockSpec((1, 1, 3, N), nb),
            pl.BlockSpec((1, 1, N, C), lambda b, t: (b, t, 0, 0)),
            pl.BlockSpec((1, 1, N, C), nb),
            pl.BlockSpec((1, NP, 1), lambda b, t: (b * T + t, 0, 0)),
            pl.BlockSpec((1, NP, 1), lambda b, t: (b * T + t, 0, 0)),
            pl.BlockSpec((1, NP, 1), lambda b, t: (b * T + t, 0, 0)),
            pl.BlockSpec((1, NP, 1), lambda b, t: (b * T + t, 0, 0)),
            pl.BlockSpec((3, C), lambda b, t: (0, 0)),
            pl.BlockSpec((C, C), lambda b, t: (0, 0)),
            pl.BlockSpec((C, C), lambda b, t: (0, 0)),
            pl.BlockSpec((1, C), lambda b, t: (0, 0)),
            pl.BlockSpec((C, 2 * C), lambda b, t: (0, 0)),
            pl.BlockSpec((1, 2 * C), lambda b, t: (0, 0)),
        ],
        out_specs=(
            pl.BlockSpec((1, 1, NP, 3), lambda b, t: (b, t, 0, 0)),
            pl.BlockSpec((1, 1, NP, 2 * C), lambda b, t: (b, t, 0, 0)),
        ),
    )(xt, xt, features, features, aidx3, ax3, ay3, az3,
      w1x, w1f, w1a, b1r, W2, b2r)
    return new_xyzs, new_feats


# speedup vs baseline: 1.0001x; 1.0001x over previous
import functools

import jax
import jax.numpy as jnp
from jax import lax
from jax.experimental import pallas as pl
from jax.experimental.pallas import tpu as pltpu
from jax.experimental.pallas import tpu_sc as plsc

B, T, N, C = 4, 4, 1024, 64
K = 16
NP = 512
F32 = jnp.float32


L = 16


def _fps_sc_body(x_hbm, y_hbm, z_hbm, aidx_hbm, ax_hbm, ay_hbm, az_hbm,
                 x_v, y_v, z_v, dmin_v, aidx_v, ax_v, ay_v, az_v):
    nc = 2
    wid = lax.axis_index("s") * nc + lax.axis_index("c")

    @pl.when(wid < B * T)
    def _():
        pltpu.sync_copy(x_hbm.at[pl.ds(wid * N, N)], x_v)
        pltpu.sync_copy(y_hbm.at[pl.ds(wid * N, N)], y_v)
        pltpu.sync_copy(z_hbm.at[pl.ds(wid * N, N)], z_v)
        lane_i = lax.iota(jnp.int32, L)
        lane_f = lane_i.astype(F32)
        big = F32(1e30)

        def fold(v, op):
            for sh in (8, 4, 2, 1):
                v = op(v, v.at[lane_i ^ sh].get(mode="promise_in_bounds"))
            return v

        for c in range(N // L):
            dmin_v[pl.ds(c * L, L)] = jnp.full((L,), 1e10, F32)

        cx0 = fold(jnp.where(lane_i == 0, x_v[pl.ds(0, L)], big), jnp.minimum)
        cy0 = fold(jnp.where(lane_i == 0, y_v[pl.ds(0, L)], big), jnp.minimum)
        cz0 = fold(jnp.where(lane_i == 0, z_v[pl.ds(0, L)], big), jnp.minimum)

        def step(i, carry):
            cx, cy, cz, farf, bA, bX, bY, bZ = carry
            sel = lane_i == (i % L)
            bA = jnp.where(sel, farf, bA)
            bX = jnp.where(sel, cx, bX)
            bY = jnp.where(sel, cy, bY)
            bZ = jnp.where(sel, cz, bZ)

            @pl.when(i % L == L - 1)
            def _flush():
                base = i - (L - 1)
                aidx_v[pl.ds(base, L)] = bA
                ax_v[pl.ds(base, L)] = bX
                ay_v[pl.ds(base, L)] = bY
                az_v[pl.ds(base, L)] = bZ

            bestv = jnp.full((L,), -1.0, F32)
            besti = jnp.full((L,), 0.0, F32)
            bestx = jnp.zeros((L,), F32)
            besty = jnp.zeros((L,), F32)
            bestz = jnp.zeros((L,), F32)
            for c in range(N // L):
                sl = pl.ds(c * L, L)
                xc = x_v[sl]
                yc = y_v[sl]
                zc = z_v[sl]
                dx = xc - cx
                dy = yc - cy
                dz = zc - cz
                d = (dx * dx + dy * dy) + dz * dz
                nm = jnp.minimum(dmin_v[sl], d)
                dmin_v[sl] = nm
                upd = nm > bestv
                bestv = jnp.where(upd, nm, bestv)
                besti = jnp.where(upd, lane_f + F32(c * L), besti)
                bestx = jnp.where(upd, xc, bestx)
                besty = jnp.where(upd, yc, besty)
                bestz = jnp.where(upd, zc, bestz)
            m = fold(bestv, jnp.maximum)
            far2 = fold(jnp.where(bestv == m, besti, F32(1e9)), jnp.minimum)
            win = besti == far2
            ncx = fold(jnp.where(win, bestx, big), jnp.minimum)
            ncy = fold(jnp.where(win, besty, big), jnp.minimum)
            ncz = fold(jnp.where(win, bestz, big), jnp.minimum)
            return (ncx, ncy, ncz, far2, bA, bX, bY, bZ)

        zz = jnp.zeros((L,), F32)
        init = (cx0, cy0, cz0, zz, zz, zz, zz, zz)
        lax.fori_loop(0, NP, step, init)

        pltpu.sync_copy(aidx_v, aidx_hbm.at[pl.ds(wid * NP, NP)])
        pltpu.sync_copy(ax_v, ax_hbm.at[pl.ds(wid * NP, NP)])
        pltpu.sync_copy(ay_v, ay_hbm.at[pl.ds(wid * NP, NP)])
        pltpu.sync_copy(az_v, az_hbm.at[pl.ds(wid * NP, NP)])


def _run_fps_sc(xyzs):
    xt = jnp.transpose(xyzs.reshape(B * T, N, 3), (2, 0, 1))
    xf = xt[0].reshape(B * T * N)
    yf = xt[1].reshape(B * T * N)
    zf = xt[2].reshape(B * T * N)
    shp = jax.ShapeDtypeStruct((B * T * NP,), F32)
    fn = pl.kernel(
        _fps_sc_body,
        out_type=(shp, shp, shp, shp),
        mesh=plsc.VectorSubcoreMesh(core_axis_name="c", subcore_axis_name="s"),
        scratch_types=[
            pltpu.VMEM((N,), F32),
            pltpu.VMEM((N,), F32),
            pltpu.VMEM((N,), F32),
            pltpu.VMEM((N,), F32),
            pltpu.VMEM((NP,), F32),
            pltpu.VMEM((NP,), F32),
            pltpu.VMEM((NP,), F32),
            pltpu.VMEM((NP,), F32),
        ],
    )
    aidx, ax, ay, az = fn(xf, yf, zf)
    rs = lambda a: a.reshape(B * T, NP)
    return rs(aidx), rs(ax), rs(ay), rs(az)


def _main_body(xta_ref, xtn_ref, fa_ref, fn_ref, aidx_ref,
               ax_ref, ay_ref, az_ref,
               w1x_ref, w1f_ref, w1a_ref, b1_ref, w2_ref, b2_ref,
               nxyz_ref, nfeat_ref):
    xa = xta_ref[0, 0]
    xn = xtn_ref[0, 0]
    f1 = fa_ref[0, 0]
    f2 = fn_ref[0, 0]
    aidx = aidx_ref[0]
    ax = ax_ref[0]
    ay = ay_ref[0]
    az = az_ref[0]
    w1x = w1x_ref[...]
    w1f = w1f_ref[...]
    w1a = w1a_ref[...]
    b1 = b1_ref[...]
    w2 = w2_ref[...]
    b2 = b2_ref[...]

    iota = lax.broadcasted_iota(jnp.int32, (NP, N), 1).astype(F32)
    oh_a = (iota == aidx).astype(F32)
    nxyz_ref[0, 0] = jnp.concatenate([ax, ay, az], axis=1)

    xyz1 = jnp.transpose(xa)
    xyz2 = jnp.transpose(xn)
    dot = functools.partial(jnp.dot, preferred_element_type=F32)
    q1 = dot(f1, w1a) - dot(xyz1, w1x)
    a_mat = dot(oh_a, q1) + b1
    p_mat = dot(f2, w1f) + dot(xyz2, w1x)

    dx = ax - xn[0:1, :]
    dy = ay - xn[1:2, :]
    dz = az - xn[2:3, :]
    s = (dx * dx + dy * dy) + dz * dz

    ones_col = jnp.ones((N, 1), F32)
    p16 = p_mat.astype(jnp.bfloat16)
    w216 = w2.astype(jnp.bfloat16)
    acc = None
    for _ in range(K):
        mval = jnp.min(s, axis=1, keepdims=True)
        eq = s == mval
        oh = jnp.where(eq, F32(1.0), F32(0.0))
        cnt = dot(oh, ones_col)
        oh16 = oh.astype(jnp.bfloat16)
        g = dot(oh16, p16) * (1.0 / cnt)
        h = jnp.maximum(g + a_mat, 0.0)
        o = jnp.maximum(dot(h.astype(jnp.bfloat16), w216) + b2, 0.0)
        acc = o if acc is None else jnp.maximum(acc, o)
        s = jnp.where(eq, F32(3e38), s)

    nfeat_ref[0, 0] = acc


def kernel(xyzs, features, W1, b1, W2, b2):
    aidx, axc, ayc, azc = _run_fps_sc(xyzs)
    aidx3 = aidx.reshape(B * T, NP, 1)
    ax3 = axc.reshape(B * T, NP, 1)
    ay3 = ayc.reshape(B * T, NP, 1)
    az3 = azc.reshape(B * T, NP, 1)

    xt = jnp.transpose(xyzs, (0, 1, 3, 2))
    w1x = W1[0:3]
    w1f = W1[3:3 + C]
    w1a = W1[3 + C:3 + 2 * C]
    b1r = b1.reshape(1, C)
    b2r = b2.reshape(1, 2 * C)

    t_last = T - 1
    nb = lambda b, t: (b, jnp.minimum(t + 1, t_last), 0, 0)

    out_shapes = (
        jax.ShapeDtypeStruct((B, T, NP, 3), F32),
        jax.ShapeDtypeStruct((B, T, NP, 2 * C), F32),
    )
    grid = (B, T)
    new_xyzs, new_feats = pl.pallas_call(
        _main_body,
        grid=grid,
        out_shape=out_shapes,
        in_specs=[
            pl.BlockSpec((1, 1, 3, N), lambda b, t: (b, t, 0, 0)),
            pl.BlockSpec((1, 1, 3, N), nb),
            pl.BlockSpec((1, 1, N, C), lambda b, t: (b, t, 0, 0)),
            pl.BlockSpec((1, 1, N, C), nb),
            pl.BlockSpec((1, NP, 1), lambda b, t: (b * T + t, 0, 0)),
            pl.BlockSpec((1, NP, 1), lambda b, t: (b * T + t, 0, 0)),
            pl.BlockSpec((1, NP, 1), lambda b, t: (b * T + t, 0, 0)),
            pl.BlockSpec((1, NP, 1), lambda b, t: (b * T + t, 0, 0)),
            pl.BlockSpec((3, C), lambda b, t: (0, 0)),
            pl.BlockSpec((C, C), lambda b, t: (0, 0)),
            pl.BlockSpec((C, C), lambda b, t: (0, 0)),
            pl.BlockSpec((1, C), lambda b, t: (0, 0)),
            pl.BlockSpec((C, 2 * C), lambda b, t: (0, 0)),
            pl.BlockSpec((1, 2 * C), lambda b, t: (0, 0)),
        ],
        out_specs=(
            pl.BlockSpec((1, 1, NP, 3), lambda b, t: (b, t, 0, 0)),
            pl.BlockSpec((1, 1, NP, 2 * C), lambda b, t: (b, t, 0, 0)),
        ),
    )(xt, xt, features, features, aidx3, ax3, ay3, az3,
      w1x, w1f, w1a, b1r, W2, b2r)
    return new_xyzs, new_feats
